# baseline (speedup 1.0000x reference)
import functools

import jax
import jax.numpy as jnp
from jax import lax
from jax.experimental import pallas as pl
from jax.experimental.pallas import tpu as pltpu
from jax.experimental.pallas import tpu_sc as plsc

D_MODEL = 128
MAX_LEN = 512
SEQ_LEN = 512
TBL = 2 * MAX_LEN - 1


def _sc_materialize(t2):
    info = plsc.get_sparse_core_info()
    nw = info.num_cores * info.num_subcores
    rows = SEQ_LEN // nw
    mesh = plsc.VectorSubcoreMesh(core_axis_name="c", subcore_axis_name="s")

    win = SEQ_LEN + rows

    SEG = SEQ_LEN // 4
    chunk_ends = (136, 272, 408, win)

    @functools.partial(
        pl.kernel,
        mesh=mesh,
        out_type=jax.ShapeDtypeStruct((SEQ_LEN, SEQ_LEN, D_MODEL), jnp.float32),
        scratch_types=[
            pltpu.VMEM((win, D_MODEL), jnp.float32),
            pltpu.SemaphoreType.DMA,
            pltpu.SemaphoreType.DMA,
            pltpu.SemaphoreType.DMA,
            pltpu.SemaphoreType.DMA,
            pltpu.SemaphoreType.DMA,
        ],
    )
    def k(t2_hbm, out_hbm, win_v, wsem, s0, s1, s2, s3):
        wid = lax.axis_index("s") * info.num_cores + lax.axis_index("c")
        base = wid * rows
        wstart = SEQ_LEN - rows - base
        stage_sems = (s0, s1, s2, s3)
        stages = []
        lo = 0
        for kk, hi in enumerate(chunk_ends):
            stages.append(
                pltpu.async_copy(
                    t2_hbm.at[pl.ds(wstart + lo, hi - lo)],
                    win_v.at[pl.ds(lo, hi - lo)],
                    stage_sems[kk],
                )
            )
            lo = hi
        writes = []
        done = set()
        for kk, hi in enumerate(chunk_ends):
            stages[kk].wait()
            for r in range(rows - 1, -1, -1):
                for s in range(4):
                    if (r, s) in done or rows - 1 - r + SEG * (s + 1) > hi:
                        continue
                    done.add((r, s))
                    writes.append(
                        pltpu.async_copy(
                            win_v.at[pl.ds(rows - 1 - r + SEG * s, SEG)],
                            out_hbm.at[base + r, pl.ds(SEG * s, SEG)],
                            wsem,
                        )
                    )
        for c in writes:
            c.wait()

    return k(t2)


def kernel(seq_len, table):
    delta = seq_len - SEQ_LEN
    t2 = table[jnp.clip(TBL - 1 + delta - jnp.arange(TBL + 1), 0, TBL - 1)]
    return _sc_materialize(t2)


# speedup vs baseline: 1.0820x; 1.0820x over previous
import functools

import jax
import jax.numpy as jnp
from jax import lax
from jax.experimental import pallas as pl
from jax.experimental.pallas import tpu as pltpu
from jax.experimental.pallas import tpu_sc as plsc

D_MODEL = 128
MAX_LEN = 512
SEQ_LEN = 512
TBL = 2 * MAX_LEN - 1


def _sc_materialize(t2):
    info = plsc.get_sparse_core_info()
    nw = info.num_cores * info.num_subcores
    rows = SEQ_LEN // nw
    mesh = plsc.VectorSubcoreMesh(core_axis_name="c", subcore_axis_name="s")

    BR, BC = 2 * rows * 2, SEQ_LEN // 4
    NCB = SEQ_LEN // BC
    win = BR + BC

    @functools.partial(
        pl.kernel,
        mesh=mesh,
        out_type=jax.ShapeDtypeStruct((SEQ_LEN, SEQ_LEN, D_MODEL), jnp.float32),
        scratch_types=[
            pltpu.VMEM((win, D_MODEL), jnp.float32),
            pltpu.SemaphoreType.DMA,
        ],
    )
    def k(t2_hbm, out_hbm, win_v, sem):
        wid = lax.axis_index("s") * info.num_cores + lax.axis_index("c")
        rb = wid // NCB
        cb = wid - rb * NCB
        i0 = rb * BR
        c0 = cb * BC
        wstart = SEQ_LEN - BR - i0 + c0
        pltpu.sync_copy(t2_hbm.at[pl.ds(wstart, win)], win_v)
        copies = []
        for r in range(BR):
            copies.append(
                pltpu.async_copy(
                    win_v.at[pl.ds(BR - 1 - r, BC)],
                    out_hbm.at[i0 + r, pl.ds(c0, BC)],
                    sem,
                )
            )
        for c in copies:
            c.wait()

    return k(t2)


def kernel(seq_len, table):
    delta = seq_len - SEQ_LEN
    t2 = table[jnp.clip(TBL - 1 + delta - jnp.arange(TBL + 1), 0, TBL - 1)]
    return _sc_materialize(t2)
